# baseline (speedup 1.0000x reference)
import functools

import jax
import jax.numpy as jnp
from jax import lax
from jax.experimental import pallas as pl
from jax.experimental.pallas import tpu as pltpu
from jax.experimental.pallas import tpu_sc as plsc

D_MODEL = 128
SELECT_N = 2048


def _score_t_body(b_ref, x_ref, w_ref, eye_ref, key_ref, p_ref):
    xb = x_ref[0]
    s = jax.lax.dot_general(
        w_ref[...], xb, (((1,), (0,)), ((), ())),
        preferred_element_type=jnp.float32,
    )
    s = s + b_ref[0]
    u = jax.lax.bitcast_convert_type(s, jnp.uint32)
    u = jnp.where(u == jnp.uint32(0x80000000), jnp.uint32(0), u)
    key = jnp.where(u >= jnp.uint32(0x80000000), u, (~u) ^ jnp.uint32(0x80000000))
    key_ref[0] = key
    p_ref[0] = jax.lax.dot_general(
        xb, eye_ref[...], (((0,), (0,)), ((), ())),
        precision=jax.lax.Precision.HIGHEST,
        preferred_element_type=jnp.float32,
    )


def _scores_and_patches(x, W, b):
    B, C, D, P = x.shape
    xf = x.reshape(B * C, D, P)
    eye = jnp.eye(D, dtype=jnp.float32)
    keys, patches = pl.pallas_call(
        _score_t_body,
        grid=(B * C,),
        in_specs=[
            pl.BlockSpec(memory_space=pltpu.SMEM),
            pl.BlockSpec((1, D, P), lambda i: (i, 0, 0)),
            pl.BlockSpec((1, D), lambda i: (0, 0)),
            pl.BlockSpec((D, D), lambda i: (0, 0)),
        ],
        out_specs=[
            pl.BlockSpec((1, 1, P), lambda i: (i, 0, 0)),
            pl.BlockSpec((1, P, D), lambda i: (i, 0, 0)),
        ],
        out_shape=[
            jax.ShapeDtypeStruct((B * C, 1, P), jnp.uint32),
            jax.ShapeDtypeStruct((B * C, P, D), jnp.float32),
        ],
    )(b, xf, W, eye)
    return keys.reshape(B, C * P), patches.reshape(B * C * P, D)


_SC_INFO = plsc.get_sparse_core_info()
_NC, _NS = _SC_INFO.num_cores, _SC_INFO.num_subcores
_NW = _NC * _NS


def _gather_rows(idx_flat, patches_flat):
    n_rows = idx_flat.shape[0]
    rows_per_w = n_rows // _NW
    mesh = plsc.VectorSubcoreMesh(core_axis_name="c", subcore_axis_name="s")

    @functools.partial(
        pl.kernel, mesh=mesh,
        out_type=jax.ShapeDtypeStruct((n_rows, D_MODEL), jnp.float32),
        scratch_types=[
            pltpu.VMEM((rows_per_w,), jnp.int32),
            pltpu.VMEM((rows_per_w, D_MODEL), jnp.float32),
            pltpu.SemaphoreType.DMA,
        ],
    )
    def k(idx_hbm, patches_hbm, out_hbm, idx_v, rows_v, sem):
        wid = lax.axis_index("s") * _NC + lax.axis_index("c")
        base = wid * rows_per_w
        pltpu.sync_copy(idx_hbm.at[pl.ds(base, rows_per_w)], idx_v)
        pltpu.async_copy(patches_hbm.at[idx_v], rows_v, sem).wait()
        pltpu.sync_copy(rows_v, out_hbm.at[pl.ds(base, rows_per_w)])

    return k(idx_flat, patches_flat)


def kernel(x, W, b):
    B, C, D, P = x.shape
    keys, patches = _scores_and_patches(x, W, b)
    top_idx = jnp.argsort(keys, axis=1, stable=True)[:, :SELECT_N]
    gidx = (top_idx + jnp.arange(B, dtype=top_idx.dtype)[:, None] * (C * P)).astype(jnp.int32)
    selected = _gather_rows(gidx.reshape(-1), patches)
    return selected.reshape(B, SELECT_N, D)
